# Initial kernel scaffold; baseline (speedup 1.0000x reference)
import jax
import jax.numpy as jnp
from jax.experimental import pallas as pl


def kernel(encoder_output, encoded_lengths, cache_rnn_state, embed, W_ih, W_hh, b_lstm, W_enc, W_pred, b_joint, W_out, b_out, label_ids):
    raise NotImplementedError("write your pallas kernel here")


# speedup vs baseline: 3.1014x; 3.1014x over previous
import jax
import jax.numpy as jnp
from jax.experimental import pallas as pl
from jax.experimental.pallas import tpu as pltpu


def _decode_body(enc_ref, len_ref, cache_ref, embed_ref, wih_ref, whh_ref,
                 blstm_ref, wenc_ref, wpred_ref, bjoint_ref, wout_ref,
                 bout_ref, labels_ref, hyps_ref, cache_out_ref, encproj_ref):
    B, T, D = enc_ref.shape
    V, H = embed_ref.shape
    HJ = wpred_ref.shape[1]

    enc = enc_ref[...].reshape(B * T, D)
    encproj_ref[...] = jnp.dot(
        enc, wenc_ref[...], preferred_element_type=jnp.float32
    ).reshape(B, T, HJ)

    lens = len_ref[...]
    blstm = blstm_ref[...]
    bjoint = bjoint_ref[...]
    bout = bout_ref[...]
    labels = labels_ref[...]

    iota_v = jax.lax.broadcasted_iota(jnp.int32, (B, V), 1)
    iota_t = jax.lax.broadcasted_iota(jnp.int32, (B, T), 1)

    h0 = cache_ref[0]
    c0 = cache_ref[1]
    prev0 = jnp.zeros((B, 1), jnp.int32)
    hyps0 = jnp.zeros((B, T), jnp.int32)

    def step(t, carry):
        h, c, prev, hyps = carry
        onehot = (iota_v == prev).astype(jnp.float32)
        x = jax.lax.dot_general(
            onehot, embed_ref[...], (((1,), (0,)), ((), ())),
            precision=jax.lax.Precision.HIGHEST,
            preferred_element_type=jnp.float32)
        gates = (jnp.dot(x, wih_ref[...], preferred_element_type=jnp.float32)
                 + jnp.dot(h, whh_ref[...], preferred_element_type=jnp.float32)
                 + blstm)
        i_g = gates[:, :H]
        f_g = gates[:, H:2 * H]
        g_g = gates[:, 2 * H:3 * H]
        o_g = gates[:, 3 * H:]
        c2 = jax.nn.sigmoid(f_g) * c + jax.nn.sigmoid(i_g) * jnp.tanh(g_g)
        h2 = jax.nn.sigmoid(o_g) * jnp.tanh(c2)
        enc_t = encproj_ref[:, pl.ds(t, 1), :].reshape(B, HJ)
        joint = jnp.tanh(
            enc_t + jnp.dot(h2, wpred_ref[...], preferred_element_type=jnp.float32)
            + bjoint)
        logits = jnp.dot(joint, wout_ref[...], preferred_element_type=jnp.float32) + bout
        m = jnp.max(logits, axis=1, keepdims=True)
        tok = jnp.min(jnp.where(logits == m, iota_v, V), axis=1, keepdims=True)
        blank = tok == 0
        h_n = jnp.where(blank, h, h2)
        c_n = jnp.where(blank, c, c2)
        prev_n = jnp.where(blank, prev, tok)
        tok_m = jnp.where(t < lens, tok, 0)
        val = jnp.sum(jnp.where(iota_v == tok_m, labels, 0), axis=1, keepdims=True)
        hyps = jnp.where(iota_t == t, val, hyps)
        return h_n, c_n, prev_n, hyps

    h, c, _, hyps = jax.lax.fori_loop(0, T, step, (h0, c0, prev0, hyps0))
    hyps_ref[...] = hyps
    cache_out_ref[0] = h
    cache_out_ref[1] = c


def kernel(encoder_output, encoded_lengths, cache_rnn_state, embed, W_ih,
           W_hh, b_lstm, W_enc, W_pred, b_joint, W_out, b_out, label_ids):
    B, T, D = encoder_output.shape
    V, H = embed.shape
    HJ = W_pred.shape[1]
    hyps, cache = pl.pallas_call(
        _decode_body,
        out_shape=(
            jax.ShapeDtypeStruct((B, T), jnp.int32),
            jax.ShapeDtypeStruct((2, B, H), jnp.float32),
        ),
        scratch_shapes=[pltpu.VMEM((B, T, HJ), jnp.float32)],
        compiler_params=pltpu.CompilerParams(
            vmem_limit_bytes=100 * 1024 * 1024),
    )(encoder_output, encoded_lengths.reshape(B, 1), cache_rnn_state, embed,
      W_ih, W_hh, b_lstm.reshape(1, -1), W_enc, W_pred,
      b_joint.reshape(1, -1), W_out, b_out.reshape(1, -1),
      label_ids.reshape(1, -1))
    return (hyps, cache)


# speedup vs baseline: 3.7143x; 1.1976x over previous
import jax
import jax.numpy as jnp
from jax.experimental import pallas as pl
from jax.experimental.pallas import tpu as pltpu


def _split3(w):
    hi = w.astype(jnp.bfloat16)
    r1 = w - hi.astype(jnp.float32)
    mid = r1.astype(jnp.bfloat16)
    lo = (r1 - mid.astype(jnp.float32)).astype(jnp.bfloat16)
    return hi, mid, lo


def _mm(a, b):
    return jax.lax.dot_general(a, b, (((1,), (0,)), ((), ())),
                               precision=jax.lax.Precision.DEFAULT,
                               preferred_element_type=jnp.float32)


def _decode_body(enc_ref, len_ref, cache_ref,
                 ehi_ref, emid_ref, elo_ref,
                 wih_ref, whh_ref, blstm_ref, wenc_ref,
                 wpred_ref, bjoint_ref, wout_ref, bout_ref, labels_ref,
                 hyps_ref, cache_out_ref, encproj_ref, g1_ref, g2_ref,
                 jp_ref, lg_ref):
    B, T, D = enc_ref.shape
    V, H = ehi_ref.shape
    HJ = wpred_ref.shape[1]

    enc = enc_ref[...].reshape(B * T, D)
    encproj_ref[...] = _mm(enc, wenc_ref[...]).reshape(B, T, HJ)

    lens = len_ref[...]
    blstm = blstm_ref[...]
    bjoint = bjoint_ref[...]
    bout = bout_ref[...]
    labels = labels_ref[...]

    iota_v = jax.lax.broadcasted_iota(jnp.int32, (B, V), 1)
    iota_t = jax.lax.broadcasted_iota(jnp.int32, (B, T), 1)

    h0 = cache_ref[0]
    c0 = cache_ref[1]
    prev0 = jnp.zeros((B, 1), jnp.int32)
    hyps0 = jnp.zeros((B, T), jnp.int32)

    def step(t, carry):
        h, c, prev, hyps = carry
        onehot = (iota_v == prev).astype(jnp.float32).astype(jnp.bfloat16)
        x = (_mm(onehot, ehi_ref[...])
             + (_mm(onehot, emid_ref[...]) + _mm(onehot, elo_ref[...])))
        g1_ref[...] = _mm(x, wih_ref[...])
        g2_ref[...] = _mm(h, whh_ref[...])
        gates = (g1_ref[...] + g2_ref[...]) + blstm
        i_g = gates[:, :H]
        f_g = gates[:, H:2 * H]
        g_g = gates[:, 2 * H:3 * H]
        o_g = gates[:, 3 * H:]
        c2 = jax.nn.sigmoid(f_g) * c + jax.nn.sigmoid(i_g) * jnp.tanh(g_g)
        h2 = jax.nn.sigmoid(o_g) * jnp.tanh(c2)
        enc_t = encproj_ref[:, pl.ds(t, 1), :].reshape(B, HJ)
        jp_ref[...] = _mm(h2, wpred_ref[...])
        joint = jnp.tanh((enc_t + jp_ref[...]) + bjoint)
        lg_ref[...] = _mm(joint, wout_ref[...])
        logits = lg_ref[...] + bout
        m = jnp.max(logits, axis=1, keepdims=True)
        tok = jnp.min(jnp.where(logits == m, iota_v, V), axis=1, keepdims=True)
        blank = tok == 0
        h_n = jnp.where(blank, h, h2)
        c_n = jnp.where(blank, c, c2)
        prev_n = jnp.where(blank, prev, tok)
        tok_m = jnp.where(t < lens, tok, 0)
        val = jnp.sum(jnp.where(iota_v == tok_m, labels, 0), axis=1, keepdims=True)
        hyps = jnp.where(iota_t == t, val, hyps)
        return h_n, c_n, prev_n, hyps

    h, c, _, hyps = jax.lax.fori_loop(0, T, step, (h0, c0, prev0, hyps0))
    hyps_ref[...] = hyps
    cache_out_ref[0] = h
    cache_out_ref[1] = c


def kernel(encoder_output, encoded_lengths, cache_rnn_state, embed, W_ih,
           W_hh, b_lstm, W_enc, W_pred, b_joint, W_out, b_out, label_ids):
    B, T, D = encoder_output.shape
    V, H = embed.shape
    HJ = W_pred.shape[1]
    e_hi, e_mid, e_lo = _split3(embed)
    hyps, cache = pl.pallas_call(
        _decode_body,
        out_shape=(
            jax.ShapeDtypeStruct((B, T), jnp.int32),
            jax.ShapeDtypeStruct((2, B, H), jnp.float32),
        ),
        scratch_shapes=[pltpu.VMEM((B, T, HJ), jnp.float32),
                        pltpu.VMEM((B, 4 * H), jnp.float32),
                        pltpu.VMEM((B, 4 * H), jnp.float32),
                        pltpu.VMEM((B, HJ), jnp.float32),
                        pltpu.VMEM((B, V), jnp.float32)],
        compiler_params=pltpu.CompilerParams(
            vmem_limit_bytes=100 * 1024 * 1024),
    )(encoder_output, encoded_lengths.reshape(B, 1), cache_rnn_state,
      e_hi, e_mid, e_lo, W_ih, W_hh,
      b_lstm.reshape(1, -1), W_enc, W_pred,
      b_joint.reshape(1, -1), W_out, b_out.reshape(1, -1),
      label_ids.reshape(1, -1))
    return (hyps, cache)


# speedup vs baseline: 4.3921x; 1.1825x over previous
import jax
import jax.numpy as jnp
from jax.experimental import pallas as pl
from jax.experimental.pallas import tpu as pltpu


def _mm(a, b):
    return jax.lax.dot_general(a, b, (((1,), (0,)), ((), ())),
                               precision=jax.lax.Precision.DEFAULT,
                               preferred_element_type=jnp.float32)


def _decode_body(enc_ref, len_ref, cache_ref, embed_ref,
                 wih_ref, whh_ref, blstm_ref, wenc_ref,
                 wpred_ref, bjoint_ref, wout_ref, bout_ref, labels_ref,
                 hyps_ref, cache_out_ref, encproj_ref, eih_ref,
                 g2_ref, jp_ref, lg_ref, pv_ref, ps_ref, dma_sem):
    B, T, D = enc_ref.shape
    V, H = embed_ref.shape
    HJ = wpred_ref.shape[1]

    enc = enc_ref[...].reshape(B * T, D)
    encproj_ref[...] = _mm(enc, wenc_ref[...]).reshape(B, T, HJ)
    eih_ref[...] = _mm(embed_ref[...], wih_ref[...])

    lens = len_ref[...]
    blstm = blstm_ref[...]
    bjoint = bjoint_ref[...]
    bout = bout_ref[...]
    labels = labels_ref[...]

    iota_v = jax.lax.broadcasted_iota(jnp.int32, (B, V), 1)
    iota_t = jax.lax.broadcasted_iota(jnp.int32, (B, T), 1)

    for b in range(B):
        ps_ref[b, 0] = 0

    h0 = cache_ref[0]
    c0 = cache_ref[1]
    prev0 = jnp.zeros((B, 1), jnp.int32)
    hyps0 = jnp.zeros((B, T), jnp.int32)

    def step(t, carry):
        h, c, prev, hyps = carry
        g1 = jnp.concatenate(
            [eih_ref[pl.ds(ps_ref[b, 0], 1), :] for b in range(B)], axis=0)
        g2_ref[...] = _mm(h, whh_ref[...])
        gates = (g1 + g2_ref[...]) + blstm
        i_g = gates[:, :H]
        f_g = gates[:, H:2 * H]
        g_g = gates[:, 2 * H:3 * H]
        o_g = gates[:, 3 * H:]
        c2 = jax.nn.sigmoid(f_g) * c + jax.nn.sigmoid(i_g) * jnp.tanh(g_g)
        h2 = jax.nn.sigmoid(o_g) * jnp.tanh(c2)
        enc_t = encproj_ref[:, pl.ds(t, 1), :].reshape(B, HJ)
        jp_ref[...] = _mm(h2, wpred_ref[...])
        joint = jnp.tanh((enc_t + jp_ref[...]) + bjoint)
        lg_ref[...] = _mm(joint, wout_ref[...])
        logits = lg_ref[...] + bout
        m = jnp.max(logits, axis=1, keepdims=True)
        tok = jnp.min(jnp.where(logits == m, iota_v, V), axis=1, keepdims=True)
        blank = tok == 0
        h_n = jnp.where(blank, h, h2)
        c_n = jnp.where(blank, c, c2)
        prev_n = jnp.where(blank, prev, tok)
        pv_ref[...] = prev_n
        pltpu.make_async_copy(pv_ref, ps_ref, dma_sem).start()
        tok_m = jnp.where(t < lens, tok, 0)
        val = jnp.sum(jnp.where(iota_v == tok_m, labels, 0), axis=1,
                      keepdims=True)
        hyps = jnp.where(iota_t == t, val, hyps)
        pltpu.make_async_copy(pv_ref, ps_ref, dma_sem).wait()
        return h_n, c_n, prev_n, hyps

    h, c, _, hyps = jax.lax.fori_loop(0, T, step, (h0, c0, prev0, hyps0))
    hyps_ref[...] = hyps
    cache_out_ref[0] = h
    cache_out_ref[1] = c


def kernel(encoder_output, encoded_lengths, cache_rnn_state, embed, W_ih,
           W_hh, b_lstm, W_enc, W_pred, b_joint, W_out, b_out, label_ids):
    B, T, D = encoder_output.shape
    V, H = embed.shape
    HJ = W_pred.shape[1]
    hyps, cache = pl.pallas_call(
        _decode_body,
        out_shape=(
            jax.ShapeDtypeStruct((B, T), jnp.int32),
            jax.ShapeDtypeStruct((2, B, H), jnp.float32),
        ),
        scratch_shapes=[pltpu.VMEM((B, T, HJ), jnp.float32),
                        pltpu.VMEM((V, 4 * H), jnp.float32),
                        pltpu.VMEM((B, 4 * H), jnp.float32),
                        pltpu.VMEM((B, HJ), jnp.float32),
                        pltpu.VMEM((B, V), jnp.float32),
                        pltpu.VMEM((B, 1), jnp.int32),
                        pltpu.SMEM((B, 1), jnp.int32),
                        pltpu.SemaphoreType.DMA],
        compiler_params=pltpu.CompilerParams(
            vmem_limit_bytes=110 * 1024 * 1024),
    )(encoder_output, encoded_lengths.reshape(B, 1), cache_rnn_state, embed,
      W_ih, W_hh, b_lstm.reshape(1, -1), W_enc, W_pred,
      b_joint.reshape(1, -1), W_out, b_out.reshape(1, -1),
      label_ids.reshape(1, -1))
    return (hyps, cache)


# speedup vs baseline: 5.5333x; 1.2598x over previous
import jax
import jax.numpy as jnp
from jax.experimental import pallas as pl
from jax.experimental.pallas import tpu as pltpu


def _mm(a, b):
    return jax.lax.dot_general(a, b, (((1,), (0,)), ((), ())),
                               precision=jax.lax.Precision.DEFAULT,
                               preferred_element_type=jnp.float32)


def _decode_body(enc_ref, len_ref, cache_ref, embed_ref,
                 wih_ref, whh_ref, blstm_ref, wenc_ref,
                 wpred_ref, bjoint_ref, wout_ref, bout_ref, labels_ref,
                 hyps_ref, cache_out_ref, encproj_ref, eih_ref,
                 g2_ref, jp_ref, lg_ref, pv_ref, ps_ref, dma_sem):
    B, T, D = enc_ref.shape
    V, H = embed_ref.shape
    HJ = wpred_ref.shape[1]

    enc = enc_ref[...].reshape(B * T, D)
    encproj_ref[...] = _mm(enc, wenc_ref[...]).reshape(B, T, HJ)
    eih_ref[...] = _mm(embed_ref[...], wih_ref[...])

    lens = len_ref[...]
    blstm = blstm_ref[...]
    bjoint = bjoint_ref[...]
    bout = bout_ref[...]
    labels = labels_ref[...]

    iota_v = jax.lax.broadcasted_iota(jnp.int32, (B, V), 1)
    iota_t = jax.lax.broadcasted_iota(jnp.int32, (B, T), 1)

    for b in range(B):
        ps_ref[b, 0] = 0

    h0 = cache_ref[0]
    c0 = cache_ref[1]
    prev0 = jnp.zeros((B, 1), jnp.int32)
    hyps0 = jnp.zeros((B, T), jnp.int32)
    g2_ref[...] = _mm(h0, whh_ref[...])
    g2_0 = g2_ref[...]

    def step(t, carry):
        h, c, prev, g2, hyps = carry
        g1 = jnp.concatenate(
            [eih_ref[pl.ds(ps_ref[b, 0], 1), :] for b in range(B)], axis=0)
        gates = (g1 + g2) + blstm
        i_g = gates[:, :H]
        f_g = gates[:, H:2 * H]
        g_g = gates[:, 2 * H:3 * H]
        o_g = gates[:, 3 * H:]
        c2 = jax.nn.sigmoid(f_g) * c + jax.nn.sigmoid(i_g) * jnp.tanh(g_g)
        h2 = jax.nn.sigmoid(o_g) * jnp.tanh(c2)
        enc_t = encproj_ref[:, pl.ds(t, 1), :].reshape(B, HJ)
        jp_ref[...] = _mm(h2, wpred_ref[...])
        joint = jnp.tanh((enc_t + jp_ref[...]) + bjoint)
        lg_ref[...] = _mm(joint, wout_ref[...])
        logits = lg_ref[...] + bout
        m = jnp.max(logits, axis=1, keepdims=True)
        tok = jnp.min(jnp.where(logits == m, iota_v, V), axis=1, keepdims=True)
        g2_ref[...] = _mm(h2, whh_ref[...])
        blank = tok == 0
        h_n = jnp.where(blank, h, h2)
        c_n = jnp.where(blank, c, c2)
        prev_n = jnp.where(blank, prev, tok)
        g2_n = jnp.where(blank, g2, g2_ref[...])
        pv_ref[...] = prev_n
        pltpu.make_async_copy(pv_ref, ps_ref, dma_sem).start()
        tok_m = jnp.where(t < lens, tok, 0)
        val = jnp.sum(jnp.where(iota_v == tok_m, labels, 0), axis=1,
                      keepdims=True)
        hyps = jnp.where(iota_t == t, val, hyps)
        pltpu.make_async_copy(pv_ref, ps_ref, dma_sem).wait()
        return h_n, c_n, prev_n, g2_n, hyps

    h, c, _, _, hyps = jax.lax.fori_loop(
        0, T, step, (h0, c0, prev0, g2_0, hyps0))
    hyps_ref[...] = hyps
    cache_out_ref[0] = h
    cache_out_ref[1] = c


def kernel(encoder_output, encoded_lengths, cache_rnn_state, embed, W_ih,
           W_hh, b_lstm, W_enc, W_pred, b_joint, W_out, b_out, label_ids):
    B, T, D = encoder_output.shape
    V, H = embed.shape
    HJ = W_pred.shape[1]
    hyps, cache = pl.pallas_call(
        _decode_body,
        out_shape=(
            jax.ShapeDtypeStruct((B, T), jnp.int32),
            jax.ShapeDtypeStruct((2, B, H), jnp.float32),
        ),
        scratch_shapes=[pltpu.VMEM((B, T, HJ), jnp.float32),
                        pltpu.VMEM((V, 4 * H), jnp.float32),
                        pltpu.VMEM((B, 4 * H), jnp.float32),
                        pltpu.VMEM((B, HJ), jnp.float32),
                        pltpu.VMEM((B, V), jnp.float32),
                        pltpu.VMEM((B, 1), jnp.int32),
                        pltpu.SMEM((B, 1), jnp.int32),
                        pltpu.SemaphoreType.DMA],
        compiler_params=pltpu.CompilerParams(
            vmem_limit_bytes=110 * 1024 * 1024),
    )(encoder_output, encoded_lengths.reshape(B, 1), cache_rnn_state, embed,
      W_ih, W_hh, b_lstm.reshape(1, -1), W_enc, W_pred,
      b_joint.reshape(1, -1), W_out, b_out.reshape(1, -1),
      label_ids.reshape(1, -1))
    return (hyps, cache)
